# baseline (speedup 1.0000x reference)
import functools

import jax
import jax.numpy as jnp
from jax.experimental import pallas as pl
from jax.experimental.pallas import tpu as pltpu

ALPHA = 0.1
EDGE_TILE = 2048


def _edge_agg_kernel(src_ref, dst_ref, wn_ref, bn_ref, ef_ref, we_ref, be_ref,
                     x_ref, o_ref, acc_ref, hs_ref, nl_ref, *, rows, te, alpha):
    j = pl.program_id(0)

    @pl.when(j == 0)
    def _():
        acc_ref[...] = jnp.zeros_like(acc_ref)
        nl = (jnp.dot(x_ref[...], wn_ref[...],
                      preferred_element_type=jnp.float32) + bn_ref[...])
        nl_ref[...] = nl.reshape(nl_ref.shape)

    f_t = (jnp.dot(we_ref[...], ef_ref[...].astype(jnp.bfloat16),
                   preferred_element_type=jnp.float32)
           + jnp.transpose(be_ref[...]))

    for mi in range(te):
        hs_ref[pl.ds(mi, 1), :] = nl_ref[src_ref[0, 0, mi]]

    msg_t = (jnp.transpose(hs_ref[...]) * f_t).astype(jnp.bfloat16)

    dst_col = jnp.transpose(dst_ref[0])
    col_ids = jax.lax.broadcasted_iota(jnp.int32, (1, rows), 1)
    onehot_t = (dst_col == col_ids).astype(jnp.bfloat16)
    acc_ref[...] += jnp.dot(msg_t, onehot_t,
                            preferred_element_type=jnp.float32)

    @pl.when(j == pl.num_programs(0) - 1)
    def _():
        o_ref[...] = x_ref[...] + alpha * jnp.maximum(
            jnp.transpose(acc_ref[...]), 0.0)


def _edge_aggregate(src, dst, wn, bn, ef_t, we_t_bf, be, x, alpha):
    n, fi = x.shape
    fe, e = ef_t.shape
    o = we_t_bf.shape[0]
    rows = n
    te = EDGE_TILE
    c = e // te
    src3 = src.reshape(c, 1, te)
    dst3 = dst.reshape(c, 1, te)
    body = functools.partial(_edge_agg_kernel, rows=rows, te=te, alpha=alpha)
    return pl.pallas_call(
        body,
        out_shape=jax.ShapeDtypeStruct((n, o), jnp.float32),
        grid=(c,),
        in_specs=[
            pl.BlockSpec((1, 1, te), lambda j: (j, 0, 0),
                         memory_space=pltpu.SMEM),
            pl.BlockSpec((1, 1, te), lambda j: (j, 0, 0)),
            pl.BlockSpec((fi, o), lambda j: (0, 0)),
            pl.BlockSpec((1, o), lambda j: (0, 0)),
            pl.BlockSpec((fe, te), lambda j: (0, j)),
            pl.BlockSpec((o, fe), lambda j: (0, 0)),
            pl.BlockSpec((1, o), lambda j: (0, 0)),
            pl.BlockSpec((rows, fi), lambda j: (0, 0)),
        ],
        out_specs=pl.BlockSpec((rows, o), lambda j: (0, 0)),
        scratch_shapes=[pltpu.VMEM((o, rows), jnp.float32),
                        pltpu.VMEM((te, o), jnp.float32),
                        pltpu.VMEM((n, 1, o), jnp.float32)],
        compiler_params=pltpu.CompilerParams(
            dimension_semantics=("arbitrary",)),
    )(src3, dst3, wn, bn.reshape(1, o), ef_t, we_t_bf, be.reshape(1, o), x)


def kernel(w_node, b_node, w_edge, b_edge, node_feats, edge_feats, src, dst):
    return _edge_aggregate(src.astype(jnp.int32), dst.astype(jnp.int32),
                           w_node, b_node, edge_feats.T,
                           w_edge.T.astype(jnp.bfloat16), b_edge,
                           node_feats, ALPHA)
